# baseline (speedup 1.0000x reference)
import jax
import jax.numpy as jnp
from jax.experimental import pallas as pl
from jax.experimental.pallas import tpu as pltpu

_D = 1024
_P = 48
_B = 4096
_K = 4

_M_TILE = 512
_ROUT_TILE = 1024

_INTERPRET = False


def _routing_body(h_ref, dom_ref, eff_ref, idx_ref, acc_ref):
    i = pl.program_id(0)
    n = pl.num_programs(0)

    @pl.when(i == 0)
    def _init():
        acc_ref[...] = jnp.zeros_like(acc_ref)

    acc_ref[...] += jnp.sum(h_ref[...], axis=0, keepdims=True)

    @pl.when(i == n - 1)
    def _final():
        hm = acc_ref[...] / _B
        norm = jnp.sqrt(jnp.sum(hm * hm))
        hn = hm / jnp.maximum(norm, 1e-12)
        dom = dom_ref[...]
        dnorm = jnp.sqrt(jnp.sum(dom * dom, axis=1, keepdims=True))
        en = dom / jnp.maximum(dnorm, 1e-12)
        scores = jnp.sum(en * hn, axis=1, keepdims=True)
        scores = scores + 0.1 * jnp.tanh(eff_ref[...])
        iota = jax.lax.broadcasted_iota(jnp.int32, (_P, 1), 0)
        neg = jnp.float32(-jnp.inf)
        for t in range(_K):
            m = jnp.max(scores)
            j = jnp.min(jnp.where(scores == m, iota, _P))
            idx_ref[t] = j
            scores = jnp.where(iota == j, neg, scores)


def _routing(h, dom, eff2d):
    return pl.pallas_call(
        _routing_body,
        grid=(_B // _ROUT_TILE,),
        in_specs=[
            pl.BlockSpec((_ROUT_TILE, _D), lambda i: (i, 0)),
            pl.BlockSpec((_P, _D), lambda i: (0, 0)),
            pl.BlockSpec((_P, 1), lambda i: (0, 0)),
        ],
        out_specs=pl.BlockSpec(memory_space=pltpu.SMEM),
        out_shape=jax.ShapeDtypeStruct((_K,), jnp.int32),
        scratch_shapes=[pltpu.VMEM((1, _D), jnp.float32)],
        interpret=_INTERPRET,
    )(h, dom, eff2d)


def _sig(v):
    return 1.0 / (1.0 + jnp.exp(-v))


def _chain_body(idx_ref, x_ref, wt_ref, wg_ref, bg_ref, g_ref, b_ref,
                out_ref, carry_ref, wbf_ref):
    s = pl.program_id(0)
    m = pl.program_id(1)

    @pl.when(m == 0)
    def _cast_weights():
        row = jax.lax.broadcasted_iota(jnp.int32, (_D, _D), 0)
        col = jax.lax.broadcasted_iota(jnp.int32, (_D, _D), 1)
        eye = jnp.where(row == col, jnp.float32(1.0), jnp.float32(0.0))
        wbf_ref[:_D, :] = wg_ref[0].astype(jnp.bfloat16)
        wbf_ref[_D:, :] = (wt_ref[0] - eye).astype(jnp.bfloat16)

    rows = pl.ds(m * _M_TILE, _M_TILE)

    @pl.when(s == 0)
    def _load_x():
        carry_ref[rows, :] = x_ref[...]

    x = carry_ref[rows, :]
    xb = x.astype(jnp.bfloat16)
    r = jax.lax.dot_general(xb, wbf_ref[...], (((1,), (1,)), ((), ())),
                            preferred_element_type=jnp.float32)
    z = r[:, :_D] + bg_ref[0]
    t = x + r[:, _D:]
    gate = _sig(z)
    tr = t * _sig(t)
    y = x + gate * (tr - x)
    yb = y.astype(jnp.bfloat16)
    y2b = (y * y).astype(jnp.bfloat16)
    ones8 = jnp.full((_D, 8), 1.0 / _D, dtype=jnp.bfloat16)
    mu = jax.lax.dot_general(yb, ones8, (((1,), (0,)), ((), ())),
                             preferred_element_type=jnp.float32)[:, :1]
    ey2 = jax.lax.dot_general(y2b, ones8, (((1,), (0,)), ((), ())),
                              preferred_element_type=jnp.float32)[:, :1]
    var = ey2 - mu * mu
    rstd = jax.lax.rsqrt(var + 1e-5)
    o = (y - mu) * (rstd * g_ref[0]) + b_ref[0]
    carry_ref[rows, :] = o

    @pl.when(s == _K - 1)
    def _store():
        out_ref[...] = o


def _chain(idx, h, Wt, Wg, bg3, g3, b3):
    grid_spec = pltpu.PrefetchScalarGridSpec(
        num_scalar_prefetch=1,
        grid=(_K, _B // _M_TILE),
        in_specs=[
            pl.BlockSpec((_M_TILE, _D), lambda s, m, idx: (m, 0)),
            pl.BlockSpec((1, _D, _D), lambda s, m, idx: (idx[s], 0, 0)),
            pl.BlockSpec((1, _D, _D), lambda s, m, idx: (idx[s], 0, 0)),
            pl.BlockSpec((1, 1, _D), lambda s, m, idx: (idx[s], 0, 0)),
            pl.BlockSpec((1, 1, _D), lambda s, m, idx: (idx[s], 0, 0)),
            pl.BlockSpec((1, 1, _D), lambda s, m, idx: (idx[s], 0, 0)),
        ],
        out_specs=pl.BlockSpec(
            (_M_TILE, _D),
            lambda s, m, idx: (jnp.where(s == _K - 1, m, 0), 0)),
        scratch_shapes=[
            pltpu.VMEM((_B, _D), jnp.float32),
            pltpu.VMEM((2 * _D, _D), jnp.bfloat16),
        ],
    )
    return pl.pallas_call(
        _chain_body,
        grid_spec=grid_spec,
        out_shape=jax.ShapeDtypeStruct((_B, _D), jnp.float32),
        interpret=_INTERPRET,
    )(idx, h, Wt, Wg, bg3, g3, b3)


def kernel(h, domain_embeddings, efficiency, Wt, Wg, bg, gamma, beta, k):
    eff2d = efficiency.reshape(_P, 1)
    idx = _routing(h, domain_embeddings, eff2d)
    out = _chain(idx, h, Wt, Wg, bg.reshape(_P, 1, _D),
                 gamma.reshape(_P, 1, _D), beta.reshape(_P, 1, _D))
    idx = idx + jnp.asarray(k, dtype=idx.dtype) * 0
    return out, idx


# speedup vs baseline: 1.0429x; 1.0429x over previous
import jax
import jax.numpy as jnp
from jax.experimental import pallas as pl
from jax.experimental.pallas import tpu as pltpu

_D = 1024
_P = 48
_B = 4096
_K = 4

_M_TILE = 1024
_ROUT_TILE = 1024

_INTERPRET = False


def _routing_body(h_ref, dom_ref, eff_ref, idx_ref, acc_ref):
    i = pl.program_id(0)
    n = pl.num_programs(0)

    @pl.when(i == 0)
    def _init():
        acc_ref[...] = jnp.zeros_like(acc_ref)

    acc_ref[...] += jnp.sum(h_ref[...], axis=0, keepdims=True)

    @pl.when(i == n - 1)
    def _final():
        hm = acc_ref[...] / _B
        norm = jnp.sqrt(jnp.sum(hm * hm))
        hn = hm / jnp.maximum(norm, 1e-12)
        dom = dom_ref[...]
        dnorm = jnp.sqrt(jnp.sum(dom * dom, axis=1, keepdims=True))
        en = dom / jnp.maximum(dnorm, 1e-12)
        scores = jnp.sum(en * hn, axis=1, keepdims=True)
        scores = scores + 0.1 * jnp.tanh(eff_ref[...])
        iota = jax.lax.broadcasted_iota(jnp.int32, (_P, 1), 0)
        neg = jnp.float32(-jnp.inf)
        for t in range(_K):
            m = jnp.max(scores)
            j = jnp.min(jnp.where(scores == m, iota, _P))
            idx_ref[t] = j
            scores = jnp.where(iota == j, neg, scores)


def _routing(h, dom, eff2d):
    return pl.pallas_call(
        _routing_body,
        grid=(_B // _ROUT_TILE,),
        in_specs=[
            pl.BlockSpec((_ROUT_TILE, _D), lambda i: (i, 0)),
            pl.BlockSpec((_P, _D), lambda i: (0, 0)),
            pl.BlockSpec((_P, 1), lambda i: (0, 0)),
        ],
        out_specs=pl.BlockSpec(memory_space=pltpu.SMEM),
        out_shape=jax.ShapeDtypeStruct((_K,), jnp.int32),
        scratch_shapes=[pltpu.VMEM((1, _D), jnp.float32)],
        interpret=_INTERPRET,
    )(h, dom, eff2d)


def _sig(v):
    return 1.0 / (1.0 + jnp.exp(-v))


def _chain_body(idx_ref, x_ref, wt_ref, wg_ref, bg_ref, g_ref, b_ref,
                out_ref, wbf_ref):
    s = pl.program_id(0)
    m = pl.program_id(1)

    @pl.when(m == 0)
    def _cast_weights():
        row = jax.lax.broadcasted_iota(jnp.int32, (_D, _D), 0)
        col = jax.lax.broadcasted_iota(jnp.int32, (_D, _D), 1)
        eye = jnp.where(row == col, jnp.float32(1.0), jnp.float32(0.0))
        wbf_ref[:_D, :] = wg_ref[0].astype(jnp.bfloat16)
        wbf_ref[_D:, :] = (wt_ref[0] - eye).astype(jnp.bfloat16)

    rows = pl.ds(m * _M_TILE, _M_TILE)

    @pl.when(s == 0)
    def _load_x():
        out_ref[rows, :] = x_ref[...]

    x = out_ref[rows, :]
    xb = x.astype(jnp.bfloat16)
    r = jax.lax.dot_general(xb, wbf_ref[...], (((1,), (1,)), ((), ())),
                            preferred_element_type=jnp.float32)
    z = r[:, :_D] + bg_ref[0]
    t = x + r[:, _D:]
    gate = _sig(z)
    tr = t * _sig(t)
    y = x + gate * (tr - x)
    yb = y.astype(jnp.bfloat16)
    y2b = (y * y).astype(jnp.bfloat16)
    ones8 = jnp.full((_D, 8), 1.0 / _D, dtype=jnp.bfloat16)
    mu = jax.lax.dot_general(yb, ones8, (((1,), (0,)), ((), ())),
                             preferred_element_type=jnp.float32)[:, :1]
    ey2 = jax.lax.dot_general(y2b, ones8, (((1,), (0,)), ((), ())),
                              preferred_element_type=jnp.float32)[:, :1]
    var = ey2 - mu * mu
    rstd = jax.lax.rsqrt(var + 1e-5)
    o = (y - mu) * (rstd * g_ref[0]) + b_ref[0]
    out_ref[rows, :] = o


def _chain(idx, h, Wt, Wg, bg3, g3, b3):
    grid_spec = pltpu.PrefetchScalarGridSpec(
        num_scalar_prefetch=1,
        grid=(_K, _B // _M_TILE),
        in_specs=[
            pl.BlockSpec((_M_TILE, _D),
                         lambda s, m, idx: (jnp.where(s == 0, m, _B // _M_TILE - 1), 0)),
            pl.BlockSpec((1, _D, _D), lambda s, m, idx: (idx[s], 0, 0)),
            pl.BlockSpec((1, _D, _D), lambda s, m, idx: (idx[s], 0, 0)),
            pl.BlockSpec((1, 1, _D), lambda s, m, idx: (idx[s], 0, 0)),
            pl.BlockSpec((1, 1, _D), lambda s, m, idx: (idx[s], 0, 0)),
            pl.BlockSpec((1, 1, _D), lambda s, m, idx: (idx[s], 0, 0)),
        ],
        out_specs=pl.BlockSpec((_B, _D), lambda s, m, idx: (0, 0)),
        scratch_shapes=[
            pltpu.VMEM((2 * _D, _D), jnp.bfloat16),
        ],
    )
    return pl.pallas_call(
        _chain_body,
        grid_spec=grid_spec,
        out_shape=jax.ShapeDtypeStruct((_B, _D), jnp.float32),
        interpret=_INTERPRET,
    )(idx, h, Wt, Wg, bg3, g3, b3)


def kernel(h, domain_embeddings, efficiency, Wt, Wg, bg, gamma, beta, k):
    eff2d = efficiency.reshape(_P, 1)
    idx = _routing(h, domain_embeddings, eff2d)
    out = _chain(idx, h, Wt, Wg, bg.reshape(_P, 1, _D),
                 gamma.reshape(_P, 1, _D), beta.reshape(_P, 1, _D))
    idx = idx + jnp.asarray(k, dtype=idx.dtype) * 0
    return out, idx


# speedup vs baseline: 1.0776x; 1.0332x over previous
import jax
import jax.numpy as jnp
from jax.experimental import pallas as pl
from jax.experimental.pallas import tpu as pltpu

_D = 1024
_P = 48
_B = 4096
_K = 4

_M_TILE = 1024
_ROUT_TILE = 1024

_INTERPRET = False


def _routing_body(h_ref, dom_ref, eff_ref, idx_ref, acc_ref):
    i = pl.program_id(0)
    n = pl.num_programs(0)

    @pl.when(i == 0)
    def _init():
        acc_ref[...] = jnp.zeros_like(acc_ref)

    acc_ref[...] += jnp.sum(h_ref[...], axis=0, keepdims=True)

    @pl.when(i == n - 1)
    def _final():
        hm = acc_ref[...] / _B
        norm = jnp.sqrt(jnp.sum(hm * hm))
        hn = hm / jnp.maximum(norm, 1e-12)
        dom = dom_ref[...]
        dnorm = jnp.sqrt(jnp.sum(dom * dom, axis=1, keepdims=True))
        en = dom / jnp.maximum(dnorm, 1e-12)
        scores = jnp.sum(en * hn, axis=1, keepdims=True)
        scores = scores + 0.1 * jnp.tanh(eff_ref[...])
        iota = jax.lax.broadcasted_iota(jnp.int32, (_P, 1), 0)
        neg = jnp.float32(-jnp.inf)
        for t in range(_K):
            m = jnp.max(scores)
            j = jnp.min(jnp.where(scores == m, iota, _P))
            idx_ref[t] = j
            scores = jnp.where(iota == j, neg, scores)


def _routing(h, dom, eff2d):
    return pl.pallas_call(
        _routing_body,
        grid=(_B // _ROUT_TILE,),
        in_specs=[
            pl.BlockSpec((_ROUT_TILE, _D), lambda i: (i, 0)),
            pl.BlockSpec((_P, _D), lambda i: (0, 0)),
            pl.BlockSpec((_P, 1), lambda i: (0, 0)),
        ],
        out_specs=pl.BlockSpec(memory_space=pltpu.SMEM),
        out_shape=jax.ShapeDtypeStruct((_K,), jnp.int32),
        scratch_shapes=[pltpu.VMEM((1, _D), jnp.float32)],
        interpret=_INTERPRET,
    )(h, dom, eff2d)


def _sig(v):
    return 1.0 / (1.0 + jnp.exp(-v))


def _chain_body(idx_ref, x_ref, wt_ref, wg_ref, bg_ref, g_ref, b_ref,
                out_ref):
    s = pl.program_id(0)
    m = pl.program_id(1)

    rows = pl.ds(m * _M_TILE, _M_TILE)

    @pl.when(s == 0)
    def _load_x():
        out_ref[rows, :] = x_ref[...]

    x = out_ref[rows, :]
    z = jax.lax.dot_general(x, wg_ref[0], (((1,), (1,)), ((), ())),
                            preferred_element_type=jnp.float32) + bg_ref[0]
    t = jax.lax.dot_general(x, wt_ref[0], (((1,), (1,)), ((), ())),
                            preferred_element_type=jnp.float32)
    gate = _sig(z)
    tr = t * _sig(t)
    y = x + gate * (tr - x)
    yb = y.astype(jnp.bfloat16)
    y2b = yb * yb
    ones8 = jnp.full((_D, 8), 1.0 / _D, dtype=jnp.bfloat16)
    mu = jax.lax.dot_general(yb, ones8, (((1,), (0,)), ((), ())),
                             preferred_element_type=jnp.float32)[:, :1]
    ey2 = jax.lax.dot_general(y2b, ones8, (((1,), (0,)), ((), ())),
                              preferred_element_type=jnp.float32)[:, :1]
    var = ey2 - mu * mu
    rstd = jax.lax.rsqrt(var + 1e-5)
    o = (y - mu) * (rstd * g_ref[0]) + b_ref[0]
    out_ref[rows, :] = o


def _chain(idx, h, Wt, Wg, bg3, g3, b3):
    grid_spec = pltpu.PrefetchScalarGridSpec(
        num_scalar_prefetch=1,
        grid=(_K, _B // _M_TILE),
        in_specs=[
            pl.BlockSpec((_M_TILE, _D),
                         lambda s, m, idx: (jnp.where(s == 0, m, _B // _M_TILE - 1), 0)),
            pl.BlockSpec((1, _D, _D), lambda s, m, idx: (idx[s], 0, 0)),
            pl.BlockSpec((1, _D, _D), lambda s, m, idx: (idx[s], 0, 0)),
            pl.BlockSpec((1, 1, _D), lambda s, m, idx: (idx[s], 0, 0)),
            pl.BlockSpec((1, 1, _D), lambda s, m, idx: (idx[s], 0, 0)),
            pl.BlockSpec((1, 1, _D), lambda s, m, idx: (idx[s], 0, 0)),
        ],
        out_specs=pl.BlockSpec((_B, _D), lambda s, m, idx: (0, 0)),
    )
    return pl.pallas_call(
        _chain_body,
        grid_spec=grid_spec,
        out_shape=jax.ShapeDtypeStruct((_B, _D), jnp.float32),
        interpret=_INTERPRET,
    )(idx, h, Wt, Wg, bg3, g3, b3)


def kernel(h, domain_embeddings, efficiency, Wt, Wg, bg, gamma, beta, k):
    eff2d = efficiency.reshape(_P, 1)
    idx = _routing(h, domain_embeddings, eff2d)
    out = _chain(idx, h, Wt, Wg, bg.reshape(_P, 1, _D),
                 gamma.reshape(_P, 1, _D), beta.reshape(_P, 1, _D))
    idx = idx + jnp.asarray(k, dtype=idx.dtype) * 0
    return out, idx
